# baseline (speedup 1.0000x reference)
import dataclasses
import functools

import jax
import jax.numpy as jnp
from jax import lax
from jax.experimental import pallas as pl
from jax.experimental.pallas import tpu as pltpu
from jax.experimental.pallas import tpu_sc as plsc

_B, _S, _D = 4096, 200, 32
_E = _B * _S
_Q = _E // 4
_NC, _NS = 2, 16
_NW = _NC * _NS
_CHUNK = 1024
_QCHUNK = _CHUNK // 4
_NCHUNK = _E // _CHUNK // _NW
_GWIN = 128
_NG = _QCHUNK // _GWIN
_TROWS = 16 * 16 * 16 * 16
_QBLK = 8192


def _quad_body(base_ref, pos_ref, wt_ref, b_ref, out_ref):
    cid = lax.broadcasted_iota(jnp.int32, (16, 1), 0)
    nid = jnp.minimum(cid // 3, 4)
    pid = cid - (cid // 3) * 3
    eb = jnp.zeros((16, _D), jnp.float32)
    for r in range(5):
        eb = eb + jnp.where(nid == r, 1.0, 0.0) * base_ref[r : r + 1, :]
    ep = jnp.zeros((16, _D), jnp.float32)
    for r in range(3):
        ep = ep + jnp.where(pid == r, 1.0, 0.0) * pos_ref[r : r + 1, :]
    e = jnp.concatenate([eb, ep], axis=1)
    fused = (
        lax.dot_general(
            e,
            wt_ref[...],
            (((1,), (1,)), ((), ())),
            preferred_element_type=jnp.float32,
        )
        + b_ref[...]
    )
    i = pl.program_id(0)
    q = i * _QBLK + lax.broadcasted_iota(jnp.int32, (_QBLK, 1), 0)
    lane = lax.broadcasted_iota(jnp.int32, (1, 16), 1)
    hots = []
    bands = []
    zero = jnp.zeros((16, _D), jnp.float32)
    for k in range(4):
        ck = (q >> (12 - 4 * k)) & 15
        hots.append(jnp.where(ck == lane, 1.0, 0.0))
        bands.append(
            jnp.concatenate(
                [zero] * k + [fused] + [zero] * (3 - k), axis=1
            )
        )
    onehot64 = jnp.concatenate(hots, axis=1)
    bd = jnp.concatenate(bands, axis=0)
    out_ref[...] = jnp.dot(onehot64, bd, preferred_element_type=jnp.float32)


def _build_quad_table(base_table, pos_table, W, b):
    return pl.pallas_call(
        _quad_body,
        grid=(_TROWS // _QBLK,),
        in_specs=[
            pl.BlockSpec((5, _D), lambda i: (0, 0)),
            pl.BlockSpec((3, _D), lambda i: (0, 0)),
            pl.BlockSpec((_D, 2 * _D), lambda i: (0, 0)),
            pl.BlockSpec((1, _D), lambda i: (0, 0)),
        ],
        out_specs=pl.BlockSpec((_QBLK, 4 * _D), lambda i: (i, 0)),
        out_shape=jax.ShapeDtypeStruct((_TROWS, 4 * _D), jnp.float32),
    )(base_table, pos_table, jnp.transpose(W), b.reshape(1, _D))


def _sc_body(nchunk, goff, tbl_hbm, n_hbm, p_hbm, out_hbm, *scratch):
    n_v = scratch[0:2]
    p_v = scratch[2:4]
    q_v = scratch[4:6]
    rows_v = scratch[6:8]
    isem = scratch[8:10]
    gsem = scratch[10:12]
    wsem = scratch[12:14]
    wid = lax.axis_index("s") * _NC + lax.axis_index("c")
    ebase = (goff + wid * nchunk) * _CHUNK
    qbase = wid * nchunk * _QCHUNK

    def idx_start(g, b):
        off = ebase + g * _CHUNK
        pltpu.async_copy(n_hbm.at[pl.ds(off, _CHUNK)], n_v[b], isem[b])
        pltpu.async_copy(p_hbm.at[pl.ds(off, _CHUNK)], p_v[b], isem[b])

    def idx_wait(b):
        pltpu.make_async_copy(n_hbm.at[pl.ds(0, _CHUNK)], n_v[b], isem[b]).wait()
        pltpu.make_async_copy(p_hbm.at[pl.ds(0, _CHUNK)], p_v[b], isem[b]).wait()

    def compute_q(b):
        nb, pb, qb = n_v[b], p_v[b], q_v[b]

        @pl.loop(0, _QCHUNK, step=16)
        def _pack(q0):
            acc = nb[pl.ds(q0, 16)] * 3 + pb[pl.ds(q0, 16)]
            for j in range(1, 4):
                s_ = pl.ds(q0 + _QCHUNK * j, 16)
                acc = acc * 16 + (nb[s_] * 3 + pb[s_])
            qb[pl.ds(q0, 16)] = acc

    def gathers_start(b):
        for j in range(_NG):
            s_ = pl.ds(j * _GWIN, _GWIN)
            pltpu.async_copy(
                tbl_hbm.at[q_v[b].at[s_]], rows_v[b].at[s_], gsem[b]
            )

    def gathers_wait(b):
        for j in range(_NG):
            s_ = pl.ds(j * _GWIN, _GWIN)
            pltpu.make_async_copy(
                tbl_hbm.at[q_v[b].at[s_]], rows_v[b].at[s_], gsem[b]
            ).wait()

    def write_start(g, b):
        off = qbase + g * _QCHUNK
        pltpu.async_copy(rows_v[b], out_hbm.at[pl.ds(off, _QCHUNK)], wsem[b])

    def write_wait(b):
        pltpu.make_async_copy(
            rows_v[b], out_hbm.at[pl.ds(0, _QCHUNK)], wsem[b]
        ).wait()

    def step(g, b):
        idx_wait(b)

        @pl.when(g >= 2)
        def _():
            write_wait(b)

        compute_q(b)
        gathers_start(b)

        @pl.when(g + 2 < nchunk)
        def _():
            idx_start(g + 2, b)

        @pl.when(g >= 1)
        def _():
            gathers_wait(b ^ 1)
            write_start(g - 1, b ^ 1)

    idx_start(0, 0)
    idx_start(1, 1)

    npairs, tail = divmod(nchunk, 2)

    @pl.loop(0, npairs)
    def _pair(gg):
        for b in range(2):
            step(gg * 2 + b, b)

    last = nchunk - 1
    if tail:
        step(last, last & 1)
    gathers_wait(last & 1)
    write_start(last, last & 1)
    write_wait(last & 1)
    write_wait((last & 1) ^ 1)


def _sc_compiler_params():
    cp = pltpu.CompilerParams()
    if "needs_layout_passes" in pltpu.CompilerParams.__dataclass_fields__:
        cp = dataclasses.replace(cp, needs_layout_passes=False)
    return cp


def _sc_lookup(tbl, n_flat, p_flat, nchunk, goff):
    mesh = plsc.VectorSubcoreMesh(core_axis_name="c", subcore_axis_name="s")
    run = pl.kernel(
        functools.partial(_sc_body, nchunk, goff),
        out_type=jax.ShapeDtypeStruct(
            (_NW * nchunk * _QCHUNK, 4 * _D), jnp.float32
        ),
        mesh=mesh,
        scratch_types=(
            [pltpu.VMEM((_CHUNK,), jnp.int32) for _ in range(2)]
            + [pltpu.VMEM((_CHUNK,), jnp.int32) for _ in range(2)]
            + [pltpu.VMEM((_QCHUNK,), jnp.int32) for _ in range(2)]
            + [pltpu.VMEM((_QCHUNK, 4 * _D), jnp.float32) for _ in range(2)]
            + [pltpu.SemaphoreType.DMA for _ in range(6)]
        ),
        compiler_params=_sc_compiler_params(),
    )
    return run(tbl, n_flat, p_flat)


_XS = 8


def _do_xpose(in_ref, out_ref):
    for s_ in range(_XS):
        xt = in_ref[s_].T
        for j in range(4):
            for k in range(4):
                out_ref[s_, :, 1024 * k + 256 * j : 1024 * k + 256 * j + 256] = (
                    xt[32 * j : 32 * (j + 1), 256 * k : 256 * (k + 1)]
                )


def _xpose_body(in_ref, out_ref):
    _do_xpose(in_ref, out_ref)


def _xpose_body_acc(in_ref, prev_ref, out_ref):
    del prev_ref
    _do_xpose(in_ref, out_ref)


def _xpose_half(out_sc_half, nslab, slab_off, prev=None):
    in3 = out_sc_half.reshape(nslab, _B // 4, 128)
    off = slab_off // _XS
    out_spec = pl.BlockSpec((_XS, _D, _B), lambda s: (s + off, 0, 0))
    out_shape = jax.ShapeDtypeStruct((_S, _D, _B), jnp.float32)
    in_spec = pl.BlockSpec((_XS, _B // 4, 128), lambda s: (s, 0, 0))
    if prev is None:
        return pl.pallas_call(
            _xpose_body,
            grid=(nslab // _XS,),
            in_specs=[in_spec],
            out_specs=out_spec,
            out_shape=out_shape,
        )(in3)
    return pl.pallas_call(
        _xpose_body_acc,
        grid=(nslab // _XS,),
        in_specs=[in_spec, pl.BlockSpec(memory_space=pl.ANY)],
        out_specs=out_spec,
        out_shape=out_shape,
        input_output_aliases={1: 0},
    )(in3, prev)


_NCH_A = 13
_NCH_B = _NCHUNK - _NCH_A
_SLAB_A = _NCH_A * _NW * _CHUNK // _B


def kernel(nucleotides, positions, base_table, pos_table, W, b):
    tbl4 = _build_quad_table(base_table, pos_table, W, b)
    n_flat = jnp.transpose(nucleotides).reshape(_E).astype(jnp.int32)
    p_flat = jnp.transpose(positions).reshape(_E).astype(jnp.int32)
    sc_a = _sc_lookup(tbl4, n_flat, p_flat, _NCH_A, 0)
    sc_b = _sc_lookup(tbl4, n_flat, p_flat, _NCH_B, _NCH_A * _NW)
    out_t = _xpose_half(sc_a, _SLAB_A, 0)
    out_t = _xpose_half(sc_b, _S - _SLAB_A, _SLAB_A, prev=out_t)
    return jnp.transpose(out_t, (2, 0, 1))


# speedup vs baseline: 1.0446x; 1.0446x over previous
import dataclasses
import functools

import jax
import jax.numpy as jnp
from jax import lax
from jax.experimental import pallas as pl
from jax.experimental.pallas import tpu as pltpu
from jax.experimental.pallas import tpu_sc as plsc

_B, _S, _D = 4096, 200, 32
_E = _B * _S
_Q = _E // 4
_NC, _NS = 2, 16
_NW = _NC * _NS
_CHUNK = 1024
_QCHUNK = _CHUNK // 4
_NCHUNK = _E // _CHUNK // _NW
_GWIN = 128
_NG = _QCHUNK // _GWIN
_TROWS = 16 * 16 * 16 * 16
_QBLK = 8192


def _quad_body(base_ref, pos_ref, wt_ref, b_ref, out_ref):
    cid = lax.broadcasted_iota(jnp.int32, (16, 1), 0)
    nid = jnp.minimum(cid // 3, 4)
    pid = cid - (cid // 3) * 3
    eb = jnp.zeros((16, _D), jnp.float32)
    for r in range(5):
        eb = eb + jnp.where(nid == r, 1.0, 0.0) * base_ref[r : r + 1, :]
    ep = jnp.zeros((16, _D), jnp.float32)
    for r in range(3):
        ep = ep + jnp.where(pid == r, 1.0, 0.0) * pos_ref[r : r + 1, :]
    e = jnp.concatenate([eb, ep], axis=1)
    fused = (
        lax.dot_general(
            e,
            wt_ref[...],
            (((1,), (1,)), ((), ())),
            preferred_element_type=jnp.float32,
        )
        + b_ref[...]
    )
    i = pl.program_id(0)
    q = i * _QBLK + lax.broadcasted_iota(jnp.int32, (_QBLK, 1), 0)
    lane = lax.broadcasted_iota(jnp.int32, (1, 16), 1)
    hots = []
    bands = []
    zero = jnp.zeros((16, _D), jnp.float32)
    for k in range(4):
        ck = (q >> (12 - 4 * k)) & 15
        hots.append(jnp.where(ck == lane, 1.0, 0.0))
        bands.append(
            jnp.concatenate(
                [zero] * k + [fused] + [zero] * (3 - k), axis=1
            )
        )
    onehot64 = jnp.concatenate(hots, axis=1)
    bd = jnp.concatenate(bands, axis=0)
    out_ref[...] = jnp.dot(onehot64, bd, preferred_element_type=jnp.float32)


def _build_quad_table(base_table, pos_table, W, b):
    return pl.pallas_call(
        _quad_body,
        grid=(_TROWS // _QBLK,),
        in_specs=[
            pl.BlockSpec((5, _D), lambda i: (0, 0)),
            pl.BlockSpec((3, _D), lambda i: (0, 0)),
            pl.BlockSpec((_D, 2 * _D), lambda i: (0, 0)),
            pl.BlockSpec((1, _D), lambda i: (0, 0)),
        ],
        out_specs=pl.BlockSpec((_QBLK, 4 * _D), lambda i: (i, 0)),
        out_shape=jax.ShapeDtypeStruct((_TROWS, 4 * _D), jnp.float32),
    )(base_table, pos_table, jnp.transpose(W), b.reshape(1, _D))


def _sc_body(nchunk, goff, tbl_hbm, n_hbm, p_hbm, out_hbm, *scratch):
    n_v = scratch[0:2]
    p_v = scratch[2:4]
    q_v = scratch[4:6]
    rows_v = scratch[6:8]
    isem = scratch[8:10]
    gsem = scratch[10:12]
    wsem = scratch[12:14]
    wid = lax.axis_index("s") * _NC + lax.axis_index("c")
    qbase = wid * nchunk * _QCHUNK

    def idx_start(g, b):
        gchunk = goff + wid * nchunk + g
        s = gchunk // 4
        i0 = (gchunk % 4) * _CHUNK
        pltpu.async_copy(n_hbm.at[s, pl.ds(i0, _CHUNK)], n_v[b], isem[b])
        pltpu.async_copy(p_hbm.at[s, pl.ds(i0, _CHUNK)], p_v[b], isem[b])

    def idx_wait(b):
        pltpu.make_async_copy(
            n_hbm.at[0, pl.ds(0, _CHUNK)], n_v[b], isem[b]
        ).wait()
        pltpu.make_async_copy(
            p_hbm.at[0, pl.ds(0, _CHUNK)], p_v[b], isem[b]
        ).wait()

    def compute_q(b):
        nb, pb, qb = n_v[b], p_v[b], q_v[b]

        @pl.loop(0, _QCHUNK, step=16)
        def _pack(q0):
            acc = nb[pl.ds(q0, 16)] * 3 + pb[pl.ds(q0, 16)]
            for j in range(1, 4):
                s_ = pl.ds(q0 + _QCHUNK * j, 16)
                acc = acc * 16 + (nb[s_] * 3 + pb[s_])
            qb[pl.ds(q0, 16)] = acc

    def gathers_start(b):
        for j in range(_NG):
            s_ = pl.ds(j * _GWIN, _GWIN)
            pltpu.async_copy(
                tbl_hbm.at[q_v[b].at[s_]], rows_v[b].at[s_], gsem[b]
            )

    def gathers_wait(b):
        for j in range(_NG):
            s_ = pl.ds(j * _GWIN, _GWIN)
            pltpu.make_async_copy(
                tbl_hbm.at[q_v[b].at[s_]], rows_v[b].at[s_], gsem[b]
            ).wait()

    def write_start(g, b):
        off = qbase + g * _QCHUNK
        pltpu.async_copy(rows_v[b], out_hbm.at[pl.ds(off, _QCHUNK)], wsem[b])

    def write_wait(b):
        pltpu.make_async_copy(
            rows_v[b], out_hbm.at[pl.ds(0, _QCHUNK)], wsem[b]
        ).wait()

    def step(g, b):
        idx_wait(b)

        @pl.when(g >= 2)
        def _():
            write_wait(b)

        compute_q(b)
        gathers_start(b)

        @pl.when(g + 2 < nchunk)
        def _():
            idx_start(g + 2, b)

        @pl.when(g >= 1)
        def _():
            gathers_wait(b ^ 1)
            write_start(g - 1, b ^ 1)

    idx_start(0, 0)
    idx_start(1, 1)

    npairs, tail = divmod(nchunk, 2)

    @pl.loop(0, npairs)
    def _pair(gg):
        for b in range(2):
            step(gg * 2 + b, b)

    last = nchunk - 1
    if tail:
        step(last, last & 1)
    gathers_wait(last & 1)
    write_start(last, last & 1)
    write_wait(last & 1)
    write_wait((last & 1) ^ 1)


def _sc_compiler_params():
    cp = pltpu.CompilerParams()
    if "needs_layout_passes" in pltpu.CompilerParams.__dataclass_fields__:
        cp = dataclasses.replace(cp, needs_layout_passes=False)
    return cp


def _sc_lookup(tbl, n_flat, p_flat, nchunk, goff):
    mesh = plsc.VectorSubcoreMesh(core_axis_name="c", subcore_axis_name="s")
    run = pl.kernel(
        functools.partial(_sc_body, nchunk, goff),
        out_type=jax.ShapeDtypeStruct(
            (_NW * nchunk * _QCHUNK, 4 * _D), jnp.float32
        ),
        mesh=mesh,
        scratch_types=(
            [pltpu.VMEM((_CHUNK,), jnp.int32) for _ in range(2)]
            + [pltpu.VMEM((_CHUNK,), jnp.int32) for _ in range(2)]
            + [pltpu.VMEM((_QCHUNK,), jnp.int32) for _ in range(2)]
            + [pltpu.VMEM((_QCHUNK, 4 * _D), jnp.float32) for _ in range(2)]
            + [pltpu.SemaphoreType.DMA for _ in range(6)]
        ),
        compiler_params=_sc_compiler_params(),
    )
    return run(tbl, n_flat, p_flat)


_XS = 8


def _do_xpose(in_ref, out_ref):
    for s_ in range(_XS):
        xt = in_ref[s_].T
        for j in range(4):
            for k in range(4):
                out_ref[s_, :, 1024 * k + 256 * j : 1024 * k + 256 * j + 256] = (
                    xt[32 * j : 32 * (j + 1), 256 * k : 256 * (k + 1)]
                )


def _xpose_body(in_ref, out_ref):
    _do_xpose(in_ref, out_ref)


def _xpose_body_acc(in_ref, prev_ref, out_ref):
    del prev_ref
    _do_xpose(in_ref, out_ref)


def _xpose_half(out_sc_half, nslab, slab_off, prev=None):
    in3 = out_sc_half.reshape(nslab, _B // 4, 128)
    off = slab_off // _XS
    out_spec = pl.BlockSpec((_XS, _D, _B), lambda s: (s + off, 0, 0))
    out_shape = jax.ShapeDtypeStruct((_S, _D, _B), jnp.float32)
    in_spec = pl.BlockSpec((_XS, _B // 4, 128), lambda s: (s, 0, 0))
    if prev is None:
        return pl.pallas_call(
            _xpose_body,
            grid=(nslab // _XS,),
            in_specs=[in_spec],
            out_specs=out_spec,
            out_shape=out_shape,
        )(in3)
    return pl.pallas_call(
        _xpose_body_acc,
        grid=(nslab // _XS,),
        in_specs=[in_spec, pl.BlockSpec(memory_space=pl.ANY)],
        out_specs=out_spec,
        out_shape=out_shape,
        input_output_aliases={1: 0},
    )(in3, prev)


_NCH_A = 13
_NCH_B = _NCHUNK - _NCH_A
_SLAB_A = _NCH_A * _NW * _CHUNK // _B


def kernel(nucleotides, positions, base_table, pos_table, W, b):
    tbl4 = _build_quad_table(base_table, pos_table, W, b)
    n_t = jnp.transpose(nucleotides).astype(jnp.int32)
    p_t = jnp.transpose(positions).astype(jnp.int32)
    sc_a = _sc_lookup(tbl4, n_t, p_t, _NCH_A, 0)
    sc_b = _sc_lookup(tbl4, n_t, p_t, _NCH_B, _NCH_A * _NW)
    out_t = _xpose_half(sc_a, _SLAB_A, 0)
    out_t = _xpose_half(sc_b, _S - _SLAB_A, _SLAB_A, prev=out_t)
    return jnp.transpose(out_t, (2, 0, 1))


# speedup vs baseline: 1.0591x; 1.0139x over previous
import dataclasses
import functools

import jax
import jax.numpy as jnp
from jax import lax
from jax.experimental import pallas as pl
from jax.experimental.pallas import tpu as pltpu
from jax.experimental.pallas import tpu_sc as plsc

_B, _S, _D = 4096, 200, 32
_E = _B * _S
_Q = _E // 4
_NC, _NS = 2, 16
_NW = _NC * _NS
_CHUNK = 1024
_QCHUNK = _CHUNK // 4
_NCHUNK = _E // _CHUNK // _NW
_GWIN = 128
_NG = _QCHUNK // _GWIN
_TROWS = 16 * 16 * 16 * 16
_QBLK = 8192


def _quad_body(base_ref, pos_ref, wt_ref, b_ref, out_ref):
    cid = lax.broadcasted_iota(jnp.int32, (16, 1), 0)
    nid = jnp.minimum(cid // 3, 4)
    pid = cid - (cid // 3) * 3
    eb = jnp.zeros((16, _D), jnp.float32)
    for r in range(5):
        eb = eb + jnp.where(nid == r, 1.0, 0.0) * base_ref[r : r + 1, :]
    ep = jnp.zeros((16, _D), jnp.float32)
    for r in range(3):
        ep = ep + jnp.where(pid == r, 1.0, 0.0) * pos_ref[r : r + 1, :]
    e = jnp.concatenate([eb, ep], axis=1)
    fused = (
        lax.dot_general(
            e,
            wt_ref[...],
            (((1,), (1,)), ((), ())),
            preferred_element_type=jnp.float32,
        )
        + b_ref[...]
    )
    i = pl.program_id(0)
    q = i * _QBLK + lax.broadcasted_iota(jnp.int32, (_QBLK, 1), 0)
    lane = lax.broadcasted_iota(jnp.int32, (1, 16), 1)
    hots = []
    bands = []
    zero = jnp.zeros((16, _D), jnp.float32)
    for k in range(4):
        ck = (q >> (12 - 4 * k)) & 15
        hots.append(jnp.where(ck == lane, 1.0, 0.0))
        bands.append(
            jnp.concatenate(
                [zero] * k + [fused] + [zero] * (3 - k), axis=1
            )
        )
    onehot64 = jnp.concatenate(hots, axis=1)
    bd = jnp.concatenate(bands, axis=0)
    out_ref[...] = jnp.dot(onehot64, bd, preferred_element_type=jnp.float32)


def _build_quad_table(base_table, pos_table, W, b):
    return pl.pallas_call(
        _quad_body,
        grid=(_TROWS // _QBLK,),
        in_specs=[
            pl.BlockSpec((5, _D), lambda i: (0, 0)),
            pl.BlockSpec((3, _D), lambda i: (0, 0)),
            pl.BlockSpec((_D, 2 * _D), lambda i: (0, 0)),
            pl.BlockSpec((1, _D), lambda i: (0, 0)),
        ],
        out_specs=pl.BlockSpec((_QBLK, 4 * _D), lambda i: (i, 0)),
        out_shape=jax.ShapeDtypeStruct((_TROWS, 4 * _D), jnp.float32),
    )(base_table, pos_table, jnp.transpose(W), b.reshape(1, _D))


def _sc_body(nchunk, goff, tbl_hbm, n_hbm, p_hbm, out_hbm, *scratch):
    n_v = scratch[0:2]
    p_v = scratch[2:4]
    q_v = scratch[4:6]
    rows_v = scratch[6:8]
    isem = scratch[8:10]
    gsem = scratch[10:12]
    wsem = scratch[12:14]
    wid = lax.axis_index("s") * _NC + lax.axis_index("c")
    qbase = wid * nchunk * _QCHUNK

    def idx_start(g, b):
        gchunk = goff + wid * nchunk + g
        s = gchunk // 4
        i0 = (gchunk % 4) * _CHUNK
        pltpu.async_copy(n_hbm.at[s, pl.ds(i0, _CHUNK)], n_v[b], isem[b])
        pltpu.async_copy(p_hbm.at[s, pl.ds(i0, _CHUNK)], p_v[b], isem[b])

    def idx_wait(b):
        pltpu.make_async_copy(
            n_hbm.at[0, pl.ds(0, _CHUNK)], n_v[b], isem[b]
        ).wait()
        pltpu.make_async_copy(
            p_hbm.at[0, pl.ds(0, _CHUNK)], p_v[b], isem[b]
        ).wait()

    def compute_q(b):
        nb, pb, qb = n_v[b], p_v[b], q_v[b]

        @pl.loop(0, _QCHUNK, step=16)
        def _pack(q0):
            acc = nb[pl.ds(q0, 16)] * 3 + pb[pl.ds(q0, 16)]
            for j in range(1, 4):
                s_ = pl.ds(q0 + _QCHUNK * j, 16)
                acc = acc * 16 + (nb[s_] * 3 + pb[s_])
            qb[pl.ds(q0, 16)] = acc

    def gathers_start(b):
        for j in range(_NG):
            s_ = pl.ds(j * _GWIN, _GWIN)
            pltpu.async_copy(
                tbl_hbm.at[q_v[b].at[s_]], rows_v[b].at[s_], gsem[b]
            )

    def gathers_wait(b):
        for j in range(_NG):
            s_ = pl.ds(j * _GWIN, _GWIN)
            pltpu.make_async_copy(
                tbl_hbm.at[q_v[b].at[s_]], rows_v[b].at[s_], gsem[b]
            ).wait()

    def write_start(g, b):
        off = qbase + g * _QCHUNK
        pltpu.async_copy(rows_v[b], out_hbm.at[pl.ds(off, _QCHUNK)], wsem[b])

    def write_wait(b):
        pltpu.make_async_copy(
            rows_v[b], out_hbm.at[pl.ds(0, _QCHUNK)], wsem[b]
        ).wait()

    def step(g, b):
        idx_wait(b)

        @pl.when(g >= 2)
        def _():
            write_wait(b)

        compute_q(b)
        gathers_start(b)

        @pl.when(g + 2 < nchunk)
        def _():
            idx_start(g + 2, b)

        @pl.when(g >= 1)
        def _():
            gathers_wait(b ^ 1)
            write_start(g - 1, b ^ 1)

    idx_start(0, 0)
    idx_start(1, 1)

    npairs, tail = divmod(nchunk, 2)

    @pl.loop(0, npairs)
    def _pair(gg):
        for b in range(2):
            step(gg * 2 + b, b)

    last = nchunk - 1
    if tail:
        step(last, last & 1)
    gathers_wait(last & 1)
    write_start(last, last & 1)
    write_wait(last & 1)
    write_wait((last & 1) ^ 1)


def _sc_compiler_params():
    cp = pltpu.CompilerParams()
    if "needs_layout_passes" in pltpu.CompilerParams.__dataclass_fields__:
        cp = dataclasses.replace(cp, needs_layout_passes=False)
    return cp


def _sc_lookup(tbl, n_flat, p_flat, nchunk, goff):
    mesh = plsc.VectorSubcoreMesh(core_axis_name="c", subcore_axis_name="s")
    run = pl.kernel(
        functools.partial(_sc_body, nchunk, goff),
        out_type=jax.ShapeDtypeStruct(
            (_NW * nchunk * _QCHUNK, 4 * _D), jnp.float32
        ),
        mesh=mesh,
        scratch_types=(
            [pltpu.VMEM((_CHUNK,), jnp.int32) for _ in range(2)]
            + [pltpu.VMEM((_CHUNK,), jnp.int32) for _ in range(2)]
            + [pltpu.VMEM((_QCHUNK,), jnp.int32) for _ in range(2)]
            + [pltpu.VMEM((_QCHUNK, 4 * _D), jnp.float32) for _ in range(2)]
            + [pltpu.SemaphoreType.DMA for _ in range(6)]
        ),
        compiler_params=_sc_compiler_params(),
    )
    return run(tbl, n_flat, p_flat)


_XS = 8


def _do_xpose(in_ref, out_ref):
    for s_ in range(_XS):
        xt = in_ref[s_].T
        for j in range(4):
            for k in range(4):
                out_ref[s_, :, 1024 * k + 256 * j : 1024 * k + 256 * j + 256] = (
                    xt[32 * j : 32 * (j + 1), 256 * k : 256 * (k + 1)]
                )


def _xpose_body(in_ref, out_ref):
    _do_xpose(in_ref, out_ref)


def _xpose_body_acc(in_ref, prev_ref, out_ref):
    del prev_ref
    _do_xpose(in_ref, out_ref)


def _xpose_half(out_sc_half, nslab, slab_off, prev=None):
    in3 = out_sc_half.reshape(nslab, _B // 4, 128)
    off = slab_off // _XS
    out_spec = pl.BlockSpec((_XS, _D, _B), lambda s: (s + off, 0, 0))
    out_shape = jax.ShapeDtypeStruct((_S, _D, _B), jnp.float32)
    in_spec = pl.BlockSpec((_XS, _B // 4, 128), lambda s: (s, 0, 0))
    if prev is None:
        return pl.pallas_call(
            _xpose_body,
            grid=(nslab // _XS,),
            in_specs=[in_spec],
            out_specs=out_spec,
            out_shape=out_shape,
        )(in3)
    return pl.pallas_call(
        _xpose_body_acc,
        grid=(nslab // _XS,),
        in_specs=[in_spec, pl.BlockSpec(memory_space=pl.ANY)],
        out_specs=out_spec,
        out_shape=out_shape,
        input_output_aliases={1: 0},
    )(in3, prev)


_NCH_A = 13
_NCH_B = _NCHUNK - _NCH_A
_SLAB_A = _NCH_A * _NW * _CHUNK // _B


def kernel(nucleotides, positions, base_table, pos_table, W, b):
    tbl4 = _build_quad_table(base_table, pos_table, W, b)
    n_t = jnp.transpose(nucleotides).astype(jnp.int32)
    p_t = jnp.transpose(positions).astype(jnp.int32)
    sc_all = _sc_lookup(tbl4, n_t, p_t, _NCHUNK, 0)
    out_t = _xpose_half(sc_all, _S, 0)
    return jnp.transpose(out_t, (2, 0, 1))
